# Initial kernel scaffold; baseline (speedup 1.0000x reference)
import functools

import jax
import jax.numpy as jnp
from jax import lax
from jax.experimental import pallas as pl
from jax.experimental.pallas import tpu as pltpu
from jax.experimental.pallas import tpu_sc as plsc

N = 10000
E = 160000
C = 256
HIDDEN = 512

NC = 2
NS = 16
NW = NC * NS
B = 128
EPW = 5120
NB = EPW // B
EPAD = NW * EPW
NP = 10240
RPS = NP // NS
CW = 64
NCH = C // CW
MW = 3 * CW


def _agg_body(x0, x1, x2, x3, srcs, dsts, wdxs, wdys, wlaps,
              outacc, outdeg,
              src_v, dst_v, wdx_v, wdy_v, wlap_v,
              gbuf, msgbuf, msgd, zbuf, acc, accdeg, gsem):
    cid = lax.axis_index("c")
    sid = lax.axis_index("s")
    wid = cid * NS + sid
    xs = [x0, x1, x2, x3]

    pltpu.sync_copy(srcs.at[wid], src_v)
    pltpu.sync_copy(dsts.at[wid], dst_v)
    pltpu.sync_copy(wdxs.at[wid], wdx_v)
    pltpu.sync_copy(wdys.at[wid], wdy_v)
    pltpu.sync_copy(wlaps.at[wid], wlap_v)

    zero16 = jnp.zeros((16,), jnp.float32)
    iota16 = lax.iota(jnp.int32, 16)

    def _zb(i, _):
        for j in range(MW // 16):
            zbuf[i, pl.ds(j * 16, 16)] = zero16
        return ()
    lax.fori_loop(0, zbuf.shape[0], _zb, ())

    def _zmd(i, _):
        flat = i * 16 + iota16
        e = flat // 8
        col = flat - e * 8
        plsc.store_scatter(msgd, [e, col], zero16)
        return ()
    lax.fori_loop(0, (B * 8) // 16, _zmd, ())

    row0 = sid * RPS

    for k in range(NCH):
        plsc.subcore_barrier()
        def _zacc(i, _):
            pltpu.sync_copy(zbuf, acc.at[pl.ds(row0 + i * zbuf.shape[0], zbuf.shape[0])])
            return ()
        lax.fori_loop(0, RPS // zbuf.shape[0], _zacc, ())
        if k == 0:
            def _zdeg(i, _):
                pltpu.sync_copy(msgd, accdeg.at[pl.ds(row0 + i * B, B)])
                return ()
            lax.fori_loop(0, RPS // B, _zdeg, ())
        plsc.subcore_barrier()

        xk = xs[k]

        def _batch(b, _):
            pltpu.async_copy(xk.at[src_v.at[b]], gbuf, gsem).wait()

            def _edge(e, _):
                gi = b * B + e
                gv = jnp.full((16,), gi, jnp.int32)
                wdx_b = plsc.load_gather(wdx_v, [gv])
                wdy_b = plsc.load_gather(wdy_v, [gv])
                wlap_b = plsc.load_gather(wlap_v, [gv])
                for r in range(CW // 16):
                    v = gbuf[e, pl.ds(r * 16, 16)]
                    msgbuf[e, pl.ds(r * 16, 16)] = wdx_b * v
                    msgbuf[e, pl.ds(CW + r * 16, 16)] = wdy_b * v
                    msgbuf[e, pl.ds(2 * CW + r * 16, 16)] = wlap_b * v
                return ()
            lax.fori_loop(0, B, _edge, ())

            pltpu.sync_copy(msgbuf, acc.at[dst_v.at[b]], add=True)

            if k == 0:
                def _grp(g, _):
                    gi0 = pl.multiple_of(b * B + g * 16, 16)
                    ev = g * 16 + iota16
                    plsc.store_scatter(msgd, [ev, jnp.zeros((16,), jnp.int32)],
                                       wdx_v[pl.ds(gi0, 16)])
                    plsc.store_scatter(msgd, [ev, jnp.ones((16,), jnp.int32)],
                                       wdy_v[pl.ds(gi0, 16)])
                    plsc.store_scatter(msgd, [ev, jnp.full((16,), 2, jnp.int32)],
                                       wlap_v[pl.ds(gi0, 16)])
                    return ()
                lax.fori_loop(0, B // 16, _grp, ())
                pltpu.sync_copy(msgd, accdeg.at[dst_v.at[b]], add=True)
            return ()
        lax.fori_loop(0, NB, _batch, ())

        plsc.subcore_barrier()
        pltpu.sync_copy(acc.at[pl.ds(row0, RPS)],
                        outacc.at[k * NC + cid, pl.ds(row0, RPS)])
        if k == 0:
            pltpu.sync_copy(accdeg.at[pl.ds(row0, RPS)],
                            outdeg.at[cid, pl.ds(row0, RPS)])


_agg = functools.partial(
    pl.kernel,
    out_type=[
        jax.ShapeDtypeStruct((NCH * NC, NP, MW), jnp.float32),
        jax.ShapeDtypeStruct((NC, NP, 8), jnp.float32),
    ],
    mesh=plsc.VectorSubcoreMesh(core_axis_name="c", subcore_axis_name="s",
                                num_cores=NC, num_subcores=NS),
    scratch_types=[
        pltpu.VMEM((NB, B), jnp.int32),
        pltpu.VMEM((NB, B), jnp.int32),
        pltpu.VMEM((EPW,), jnp.float32),
        pltpu.VMEM((EPW,), jnp.float32),
        pltpu.VMEM((EPW,), jnp.float32),
        pltpu.VMEM((B, CW), jnp.float32),
        pltpu.VMEM((B, MW), jnp.float32),
        pltpu.VMEM((B, 8), jnp.float32),
        pltpu.VMEM((64, MW), jnp.float32),
        pltpu.VMEM_SHARED((NP, MW), jnp.float32),
        pltpu.VMEM_SHARED((NP, 8), jnp.float32),
        pltpu.SemaphoreType.DMA,
    ],
)(_agg_body)


BLK = 128


def _mlp_body(hc_ref, x_ref, acc_ref, deg_ref,
              W1_ref, b1_ref, g1_ref, bt1_ref,
              W2_ref, b2_ref, g2_ref, bt2_ref,
              W3_ref, b3_ref, o_ref):
    h = hc_ref[0, 0]
    xb = x_ref[...]
    degb = deg_ref[...]
    degs = degb[0] + degb[1]

    inv_sqrt2 = 0.7071067811865476

    def gelu(v):
        return 0.5 * v * (1.0 + lax.erf(v * inv_sqrt2))

    def ln(v, g, b):
        mu = jnp.mean(v, axis=-1, keepdims=True)
        var = jnp.mean((v - mu) ** 2, axis=-1, keepdims=True)
        return (v - mu) * lax.rsqrt(var + 1e-5) * g + b

    hs = [h, h, h * h]
    acct = acc_ref[...]
    z = xb @ W1_ref[0:C]
    for kcoef in range(3):
        cols = [acct[2 * ch, :, kcoef * CW:(kcoef + 1) * CW]
                + acct[2 * ch + 1, :, kcoef * CW:(kcoef + 1) * CW]
                for ch in range(NCH)]
        Y = jnp.concatenate(cols, axis=1)
        feat = (Y - degs[:, kcoef:kcoef + 1] * xb) * hs[kcoef]
        z = z + feat @ W1_ref[(kcoef + 1) * C:(kcoef + 2) * C]
    z = z + b1_ref[...]
    z = gelu(ln(z, g1_ref[...], bt1_ref[...]))
    z = z @ W2_ref[...] + b2_ref[...]
    z = gelu(ln(z, g2_ref[...], bt2_ref[...]))
    o_ref[...] = z @ W3_ref[...] + b3_ref[...] + xb


def kernel(x, edge_index, coeff_dx, coeff_dy, coeff_lap, h_char,
           W1, b1, g1, bt1, W2, b2, g2, bt2, W3, b3):
    src = edge_index[0]
    dst = edge_index[1]
    pad = EPAD - E
    srcp = jnp.concatenate([src, jnp.zeros((pad,), jnp.int32)]).reshape(NW, NB, B)
    dstp = jnp.concatenate([dst, jnp.zeros((pad,), jnp.int32)]).reshape(NW, NB, B)
    zpadf = jnp.zeros((pad,), jnp.float32)
    wdx = jnp.concatenate([coeff_dx.reshape(E), zpadf]).reshape(NW, EPW)
    wdy = jnp.concatenate([coeff_dy.reshape(E), zpadf]).reshape(NW, EPW)
    wlap = jnp.concatenate([coeff_lap.reshape(E), zpadf]).reshape(NW, EPW)

    xc = x.reshape(N, NCH, CW).transpose(1, 0, 2)

    outacc, outdeg = _agg(xc[0], xc[1], xc[2], xc[3],
                          srcp, dstp, wdx, wdy, wlap)

    xp = jnp.pad(x, ((0, NP - N), (0, 0)))
    hc = h_char.reshape(1, 1)

    grid = (NP // BLK,)
    out = pl.pallas_call(
        _mlp_body,
        grid=grid,
        in_specs=[
            pl.BlockSpec(memory_space=pltpu.SMEM),
            pl.BlockSpec((BLK, C), lambda i: (i, 0)),
            pl.BlockSpec((NCH * NC, BLK, MW), lambda i: (0, i, 0)),
            pl.BlockSpec((NC, BLK, 8), lambda i: (0, i, 0)),
            pl.BlockSpec((4 * C, HIDDEN), lambda i: (0, 0)),
            pl.BlockSpec((1, HIDDEN), lambda i: (0, 0)),
            pl.BlockSpec((1, HIDDEN), lambda i: (0, 0)),
            pl.BlockSpec((1, HIDDEN), lambda i: (0, 0)),
            pl.BlockSpec((HIDDEN, C), lambda i: (0, 0)),
            pl.BlockSpec((1, C), lambda i: (0, 0)),
            pl.BlockSpec((1, C), lambda i: (0, 0)),
            pl.BlockSpec((1, C), lambda i: (0, 0)),
            pl.BlockSpec((C, C), lambda i: (0, 0)),
            pl.BlockSpec((1, C), lambda i: (0, 0)),
        ],
        out_specs=pl.BlockSpec((BLK, C), lambda i: (i, 0)),
        out_shape=jax.ShapeDtypeStruct((NP, C), jnp.float32),
    )(hc, xp, outacc, outdeg,
      W1, b1.reshape(1, HIDDEN), g1.reshape(1, HIDDEN), bt1.reshape(1, HIDDEN),
      W2, b2.reshape(1, C), g2.reshape(1, C), bt2.reshape(1, C),
      W3, b3.reshape(1, C))
    return out[:N]


# speedup vs baseline: 1.0544x; 1.0544x over previous
import functools

import jax
import jax.numpy as jnp
from jax import lax
from jax.experimental import pallas as pl
from jax.experimental.pallas import tpu as pltpu
from jax.experimental.pallas import tpu_sc as plsc

N = 10000
E = 160000
C = 256
HIDDEN = 512

NC = 2
NS = 16
NW = NC * NS
B = 128
EPW = 5120
NB = EPW // B
EPAD = NW * EPW
NP = 10240
RPS = NP // NS
HW = 128
NPH = 7


def _agg_body(xh0, xh1, srcs, dsts, wdxs, wdys, wlaps,
              outacc,
              src_v, dst_v, wb_v,
              gbuf, msgbuf, zbuf, acc, gsem):
    cid = lax.axis_index("c")
    sid = lax.axis_index("s")
    wid = cid * NS + sid

    pltpu.sync_copy(srcs.at[wid], src_v)
    pltpu.sync_copy(dsts.at[wid], dst_v)

    zero16 = jnp.zeros((16,), jnp.float32)
    iota16 = lax.iota(jnp.int32, 16)

    _dnums = lax.GatherDimensionNumbers(
        offset_dims=(), collapsed_slice_dims=(0,), start_index_map=(0,))

    def _bcast(vec16, t):
        idx = jnp.full((16, 1), t, jnp.int32)
        return lax.gather(vec16, idx, _dnums, (1,),
                          mode=lax.GatherScatterMode.PROMISE_IN_BOUNDS)

    def _zb(i, _):
        for j in range(HW // 16):
            zbuf[i, pl.ds(j * 16, 16)] = zero16
        return ()
    lax.fori_loop(0, zbuf.shape[0], _zb, ())

    row0 = sid * RPS

    def _zero_acc():
        def _zacc(i, _):
            pltpu.sync_copy(zbuf, acc.at[pl.ds(row0 + i * zbuf.shape[0], zbuf.shape[0])])
            return ()
        lax.fori_loop(0, RPS // zbuf.shape[0], _zacc, ())

    def _dump(p):
        pltpu.sync_copy(acc.at[pl.ds(row0, RPS)],
                        outacc.at[p * NC + cid, pl.ds(row0, RPS)])

    p = 0
    for half in range(2):
        xk = xh0 if half == 0 else xh1
        for w_hbm in (wdxs, wdys, wlaps):
            plsc.subcore_barrier()
            _zero_acc()
            plsc.subcore_barrier()

            def _batch(b, _, w_hbm=w_hbm, xk=xk):
                pltpu.sync_copy(w_hbm.at[wid, b], wb_v.at[0])
                pltpu.async_copy(xk.at[src_v.at[b]], gbuf, gsem).wait()

                def _grp16(g, _):
                    gi0 = pl.multiple_of(g * 16, 16)
                    w16 = wb_v[0, pl.ds(gi0, 16)]

                    def _e2(t, _):
                        e = g * 16 + t
                        w_b = _bcast(w16, t)
                        for r in range(HW // 16):
                            msgbuf[e, pl.ds(r * 16, 16)] = w_b * gbuf[e, pl.ds(r * 16, 16)]
                        return ()
                    lax.fori_loop(0, 16, _e2, ())
                    return ()
                lax.fori_loop(0, B // 16, _grp16, ())

                pltpu.sync_copy(msgbuf, acc.at[dst_v.at[b]], add=True)
                return ()
            lax.fori_loop(0, NB, _batch, ())

            plsc.subcore_barrier()
            _dump(p)
            p += 1

    plsc.subcore_barrier()
    _zero_acc()
    plsc.subcore_barrier()

    def _zmsg(e, _):
        for j in range(HW // 16):
            msgbuf[e, pl.ds(j * 16, 16)] = zero16
        return ()
    lax.fori_loop(0, B, _zmsg, ())

    lane0 = iota16 == 0
    lane1 = iota16 == 1
    lane2 = iota16 == 2

    def _dbatch(b, _):
        pltpu.sync_copy(wdxs.at[wid, b], wb_v.at[0])
        pltpu.sync_copy(wdys.at[wid, b], wb_v.at[1])
        pltpu.sync_copy(wlaps.at[wid, b], wb_v.at[2])

        def _dgrp(g, _):
            gi0 = pl.multiple_of(g * 16, 16)
            wdx16 = wb_v[0, pl.ds(gi0, 16)]
            wdy16 = wb_v[1, pl.ds(gi0, 16)]
            wlap16 = wb_v[2, pl.ds(gi0, 16)]

            def _de(t, _):
                e = g * 16 + t
                v = jnp.where(lane0, _bcast(wdx16, t), zero16)
                v = jnp.where(lane1, _bcast(wdy16, t), v)
                v = jnp.where(lane2, _bcast(wlap16, t), v)
                msgbuf[e, pl.ds(0, 16)] = v
                return ()
            lax.fori_loop(0, 16, _de, ())
            return ()
        lax.fori_loop(0, B // 16, _dgrp, ())
        pltpu.sync_copy(msgbuf, acc.at[dst_v.at[b]], add=True)
        return ()
    lax.fori_loop(0, NB, _dbatch, ())

    plsc.subcore_barrier()
    _dump(6)


_agg = functools.partial(
    pl.kernel,
    out_type=[
        jax.ShapeDtypeStruct((NPH * NC, NP, HW), jnp.float32),
    ],
    mesh=plsc.VectorSubcoreMesh(core_axis_name="c", subcore_axis_name="s",
                                num_cores=NC, num_subcores=NS),
    scratch_types=[
        pltpu.VMEM((NB, B), jnp.int32),
        pltpu.VMEM((NB, B), jnp.int32),
        pltpu.VMEM((3, B), jnp.float32),
        pltpu.VMEM((B, HW), jnp.float32),
        pltpu.VMEM((B, HW), jnp.float32),
        pltpu.VMEM((32, HW), jnp.float32),
        pltpu.VMEM_SHARED((NP, HW), jnp.float32),
        pltpu.SemaphoreType.DMA,
    ],
)(_agg_body)


BLK = 128


def _mlp_body(hc_ref, x_ref, acc_ref,
              W1_ref, b1_ref, g1_ref, bt1_ref,
              W2_ref, b2_ref, g2_ref, bt2_ref,
              W3_ref, b3_ref, o_ref):
    h = hc_ref[0, 0]
    xb = x_ref[...]
    acct = acc_ref[...]
    degc = acct[12] + acct[13]

    inv_sqrt2 = 0.7071067811865476

    def gelu(v):
        return 0.5 * v * (1.0 + lax.erf(v * inv_sqrt2))

    def ln(v, g, b):
        mu = jnp.mean(v, axis=-1, keepdims=True)
        var = jnp.mean((v - mu) ** 2, axis=-1, keepdims=True)
        return (v - mu) * lax.rsqrt(var + 1e-5) * g + b

    hs = [h, h, h * h]
    z = xb @ W1_ref[0:C]
    for kc in range(3):
        Y = jnp.concatenate(
            [acct[2 * kc] + acct[2 * kc + 1],
             acct[6 + 2 * kc] + acct[6 + 2 * kc + 1]], axis=1)
        deg = degc[:, kc:kc + 1]
        feat = (Y - deg * xb) * hs[kc]
        z = z + feat @ W1_ref[(kc + 1) * C:(kc + 2) * C]
    z = z + b1_ref[...]
    z = gelu(ln(z, g1_ref[...], bt1_ref[...]))
    z = z @ W2_ref[...] + b2_ref[...]
    z = gelu(ln(z, g2_ref[...], bt2_ref[...]))
    o_ref[...] = z @ W3_ref[...] + b3_ref[...] + xb


def kernel(x, edge_index, coeff_dx, coeff_dy, coeff_lap, h_char,
           W1, b1, g1, bt1, W2, b2, g2, bt2, W3, b3):
    src = edge_index[0]
    dst = edge_index[1]
    pad = EPAD - E
    srcp = jnp.concatenate([src, jnp.zeros((pad,), jnp.int32)]).reshape(NW, NB, B)
    dstp = jnp.concatenate([dst, jnp.zeros((pad,), jnp.int32)]).reshape(NW, NB, B)
    zpadf = jnp.zeros((pad,), jnp.float32)
    wdx = jnp.concatenate([coeff_dx.reshape(E), zpadf]).reshape(NW, NB, B)
    wdy = jnp.concatenate([coeff_dy.reshape(E), zpadf]).reshape(NW, NB, B)
    wlap = jnp.concatenate([coeff_lap.reshape(E), zpadf]).reshape(NW, NB, B)

    xh0 = jnp.asarray(x[:, :HW])
    xh1 = jnp.asarray(x[:, HW:])

    (outacc,) = _agg(xh0, xh1, srcp, dstp, wdx, wdy, wlap)

    xp = jnp.pad(x, ((0, NP - N), (0, 0)))
    hc = h_char.reshape(1, 1)

    grid = (NP // BLK,)
    out = pl.pallas_call(
        _mlp_body,
        grid=grid,
        in_specs=[
            pl.BlockSpec(memory_space=pltpu.SMEM),
            pl.BlockSpec((BLK, C), lambda i: (i, 0)),
            pl.BlockSpec((NPH * NC, BLK, HW), lambda i: (0, i, 0)),
            pl.BlockSpec((4 * C, HIDDEN), lambda i: (0, 0)),
            pl.BlockSpec((1, HIDDEN), lambda i: (0, 0)),
            pl.BlockSpec((1, HIDDEN), lambda i: (0, 0)),
            pl.BlockSpec((1, HIDDEN), lambda i: (0, 0)),
            pl.BlockSpec((HIDDEN, C), lambda i: (0, 0)),
            pl.BlockSpec((1, C), lambda i: (0, 0)),
            pl.BlockSpec((1, C), lambda i: (0, 0)),
            pl.BlockSpec((1, C), lambda i: (0, 0)),
            pl.BlockSpec((C, C), lambda i: (0, 0)),
            pl.BlockSpec((1, C), lambda i: (0, 0)),
        ],
        out_specs=pl.BlockSpec((BLK, C), lambda i: (i, 0)),
        out_shape=jax.ShapeDtypeStruct((NP, C), jnp.float32),
    )(hc, xp, outacc,
      W1, b1.reshape(1, HIDDEN), g1.reshape(1, HIDDEN), bt1.reshape(1, HIDDEN),
      W2, b2.reshape(1, C), g2.reshape(1, C), bt2.reshape(1, C),
      W3, b3.reshape(1, C))
    return out[:N]


# speedup vs baseline: 1.5909x; 1.5087x over previous
import functools

import jax
import jax.numpy as jnp
from jax import lax
from jax.experimental import pallas as pl
from jax.experimental.pallas import tpu as pltpu
from jax.experimental.pallas import tpu_sc as plsc

N = 10000
E = 160000
C = 256
HIDDEN = 512

NC = 2
NS = 16
NW = NC * NS
B = 64
EPW = 5120
NB = EPW // B
EPAD = NW * EPW
NP = 10240
RPS = NP // NS
HW = 128
NPH = 7
ZR = 32
NZ = RPS // ZR


def _agg_body(xh, srcs, dsts, wcat, zeros,
              outacc,
              src_v, dst_v, wb, gbuf, msgbuf, acc,
              gsem0, gsem1, ssem0, ssem1):
    cid = lax.axis_index("c")
    sid = lax.axis_index("s")
    wid = cid * NS + sid

    pltpu.sync_copy(srcs.at[wid], src_v)
    pltpu.sync_copy(dsts.at[wid], dst_v)

    zero16 = jnp.zeros((16,), jnp.float32)
    iota16 = lax.iota(jnp.int32, 16)

    _dnums = lax.GatherDimensionNumbers(
        offset_dims=(), collapsed_slice_dims=(0,), start_index_map=(0,))

    def _bcast(vec16, t):
        idx = jnp.full((16, 1), t, jnp.int32)
        return lax.gather(vec16, idx, _dnums, (1,),
                          mode=lax.GatherScatterMode.PROMISE_IN_BOUNDS)

    row0 = sid * RPS
    gsems = (gsem0, gsem1)
    ssems = (ssem0, ssem1)

    def _zero_acc():
        pltpu.sync_copy(zeros, acc.at[pl.ds(row0, RPS)])

    def _dump(p):
        pltpu.sync_copy(acc.at[pl.ds(row0, RPS)],
                        outacc.at[p * NC + cid, pl.ds(row0, RPS)])

    def _issue_gather(p, g, b, j):
        half = p // 3
        cf = p - 3 * half
        pltpu.async_copy(xh.at[half].at[src_v.at[g, pl.ds(j * B, B)]],
                         gbuf.at[j], gsems[j])
        pltpu.async_copy(wcat.at[cf, wid, b], wb.at[j, pl.ds(0, B)], gsems[j])

    def _wait_gather(j):
        pltpu.make_async_copy(xh.at[0].at[src_v.at[0, pl.ds(0, B)]],
                              gbuf.at[j], gsems[j]).wait()
        pltpu.make_async_copy(wcat.at[0, 0, 0], wb.at[j, pl.ds(0, B)], gsems[j]).wait()

    def _compute(j):
        def _grp(gx, _):
            base = gx * 16
            w16 = wb[j, pl.ds(gx * 16, 16)]

            def _e2(t, _):
                e = base + t
                w_b = _bcast(w16, t)
                for r in range(HW // 16):
                    msgbuf[j, e, pl.ds(r * 16, 16)] = w_b * gbuf[j, e, pl.ds(r * 16, 16)]
                return ()
            lax.fori_loop(0, 16, _e2, ())
            return ()
        lax.fori_loop(0, B // 16, _grp, ())

    def _issue_scatter(b, j):
        pltpu.async_copy(msgbuf.at[j], acc.at[dst_v.at[b]], ssems[j], add=True)

    def _wait_scatter(j):
        pltpu.make_async_copy(msgbuf.at[j], acc.at[pl.ds(0, B)], ssems[j]).wait()

    def _phase(p, _):
        plsc.subcore_barrier()
        _zero_acc()
        plsc.subcore_barrier()

        _issue_gather(p, 0, 0, 0)

        def _pair(g, _):
            b0 = g * 2
            b1 = b0 + 1
            _wait_gather(0)
            _issue_gather(p, g, b1, 1)

            @pl.when(b0 >= 2)
            def _():
                _wait_scatter(0)
            _compute(0)
            _issue_scatter(b0, 0)
            _wait_gather(1)

            @pl.when(b1 + 1 < NB)
            def _():
                _issue_gather(p, g + 1, b1 + 1, 0)

            @pl.when(b1 >= 2)
            def _():
                _wait_scatter(1)
            _compute(1)
            _issue_scatter(b1, 1)
            return ()
        lax.fori_loop(0, NB // 2, _pair, ())

        _wait_scatter(0)
        _wait_scatter(1)
        plsc.subcore_barrier()
        _dump(p)
        return ()
    lax.fori_loop(0, 6, _phase, ())

    plsc.subcore_barrier()
    _zero_acc()
    plsc.subcore_barrier()

    def _zmsg(e, _):
        for j in range(HW // 16):
            msgbuf[0, e, pl.ds(j * 16, 16)] = zero16
            msgbuf[1, e, pl.ds(j * 16, 16)] = zero16
        return ()
    lax.fori_loop(0, B, _zmsg, ())

    lane0 = iota16 == 0
    lane1 = iota16 == 1
    lane2 = iota16 == 2

    def _issue_w3(b, j):
        for kc in range(3):
            pltpu.async_copy(wcat.at[kc, wid, b], wb.at[j, pl.ds(kc * B, B)], gsems[j])

    def _wait_w3(j):
        for kc in range(3):
            pltpu.make_async_copy(wcat.at[0, 0, 0], wb.at[j, pl.ds(kc * B, B)],
                                  gsems[j]).wait()

    def _dcompute(j):
        def _dgrp(gx, _):
            base = gx * 16
            wdx16 = wb[j, pl.ds(gx * 16, 16)]
            wdy16 = wb[j, pl.ds(B + gx * 16, 16)]
            wlap16 = wb[j, pl.ds(2 * B + gx * 16, 16)]

            def _de(t, _):
                e = base + t
                v = jnp.where(lane0, _bcast(wdx16, t), zero16)
                v = jnp.where(lane1, _bcast(wdy16, t), v)
                v = jnp.where(lane2, _bcast(wlap16, t), v)
                msgbuf[j, e, pl.ds(0, 16)] = v
                return ()
            lax.fori_loop(0, 16, _de, ())
            return ()
        lax.fori_loop(0, B // 16, _dgrp, ())

    _issue_w3(0, 0)

    def _dpair(g, _):
        b0 = g * 2
        b1 = b0 + 1
        _wait_w3(0)
        _issue_w3(b1, 1)

        @pl.when(b0 >= 2)
        def _():
            _wait_scatter(0)
        _dcompute(0)
        _issue_scatter(b0, 0)

        _wait_w3(1)

        @pl.when(b1 + 1 < NB)
        def _():
            _issue_w3(b1 + 1, 0)

        @pl.when(b1 >= 2)
        def _():
            _wait_scatter(1)
        _dcompute(1)
        _issue_scatter(b1, 1)
        return ()
    lax.fori_loop(0, NB // 2, _dpair, ())

    _wait_scatter(0)
    _wait_scatter(1)
    plsc.subcore_barrier()
    _dump(6)


_agg = functools.partial(
    pl.kernel,
    out_type=[
        jax.ShapeDtypeStruct((NPH * NC, NP, HW), jnp.float32),
    ],
    mesh=plsc.VectorSubcoreMesh(core_axis_name="c", subcore_axis_name="s",
                                num_cores=NC, num_subcores=NS),
    scratch_types=[
        pltpu.VMEM((NB // 2, 2 * B), jnp.int32),
        pltpu.VMEM((NB, B), jnp.int32),
        pltpu.VMEM((2, 3 * B), jnp.float32),
        pltpu.VMEM((2, B, HW), jnp.float32),
        pltpu.VMEM((2, B, HW), jnp.float32),
        pltpu.VMEM_SHARED((NP, HW), jnp.float32),
        pltpu.SemaphoreType.DMA,
        pltpu.SemaphoreType.DMA,
        pltpu.SemaphoreType.DMA,
        pltpu.SemaphoreType.DMA,
    ],
)(_agg_body)


BLK = 128


def _mlp_body(hc_ref, x_ref, acc_ref,
              W1_ref, b1_ref, g1_ref, bt1_ref,
              W2_ref, b2_ref, g2_ref, bt2_ref,
              W3_ref, b3_ref, o_ref):
    h = hc_ref[0, 0]
    xb = x_ref[...]
    acct = acc_ref[...]
    degc = acct[12] + acct[13]

    inv_sqrt2 = 0.7071067811865476

    def gelu(v):
        return 0.5 * v * (1.0 + lax.erf(v * inv_sqrt2))

    def ln(v, g, b):
        mu = jnp.mean(v, axis=-1, keepdims=True)
        var = jnp.mean((v - mu) ** 2, axis=-1, keepdims=True)
        return (v - mu) * lax.rsqrt(var + 1e-5) * g + b

    hs = [h, h, h * h]
    z = xb @ W1_ref[0:C]
    for kc in range(3):
        Y = jnp.concatenate(
            [acct[2 * kc] + acct[2 * kc + 1],
             acct[6 + 2 * kc] + acct[6 + 2 * kc + 1]], axis=1)
        deg = degc[:, kc:kc + 1]
        feat = (Y - deg * xb) * hs[kc]
        z = z + feat @ W1_ref[(kc + 1) * C:(kc + 2) * C]
    z = z + b1_ref[...]
    z = gelu(ln(z, g1_ref[...], bt1_ref[...]))
    z = z @ W2_ref[...] + b2_ref[...]
    z = gelu(ln(z, g2_ref[...], bt2_ref[...]))
    o_ref[...] = z @ W3_ref[...] + b3_ref[...] + xb


def kernel(x, edge_index, coeff_dx, coeff_dy, coeff_lap, h_char,
           W1, b1, g1, bt1, W2, b2, g2, bt2, W3, b3):
    src = edge_index[0]
    dst = edge_index[1]
    pad = EPAD - E
    srcp = jnp.concatenate([src, jnp.zeros((pad,), jnp.int32)]).reshape(NW, NB // 2, 2 * B)
    dstp = jnp.concatenate([dst, jnp.zeros((pad,), jnp.int32)]).reshape(NW, NB, B)
    zpadf = jnp.zeros((pad,), jnp.float32)
    wcat = jnp.stack([
        jnp.concatenate([coeff_dx.reshape(E), zpadf]).reshape(NW, NB, B),
        jnp.concatenate([coeff_dy.reshape(E), zpadf]).reshape(NW, NB, B),
        jnp.concatenate([coeff_lap.reshape(E), zpadf]).reshape(NW, NB, B),
    ])

    xh = jnp.stack([x[:, :HW], x[:, HW:]])
    zeros = jnp.zeros((RPS, HW), jnp.float32)

    (outacc,) = _agg(xh, srcp, dstp, wcat, zeros)

    xp = jnp.pad(x, ((0, NP - N), (0, 0)))
    hc = h_char.reshape(1, 1)

    grid = (NP // BLK,)
    out = pl.pallas_call(
        _mlp_body,
        grid=grid,
        in_specs=[
            pl.BlockSpec(memory_space=pltpu.SMEM),
            pl.BlockSpec((BLK, C), lambda i: (i, 0)),
            pl.BlockSpec((NPH * NC, BLK, HW), lambda i: (0, i, 0)),
            pl.BlockSpec((4 * C, HIDDEN), lambda i: (0, 0)),
            pl.BlockSpec((1, HIDDEN), lambda i: (0, 0)),
            pl.BlockSpec((1, HIDDEN), lambda i: (0, 0)),
            pl.BlockSpec((1, HIDDEN), lambda i: (0, 0)),
            pl.BlockSpec((HIDDEN, C), lambda i: (0, 0)),
            pl.BlockSpec((1, C), lambda i: (0, 0)),
            pl.BlockSpec((1, C), lambda i: (0, 0)),
            pl.BlockSpec((1, C), lambda i: (0, 0)),
            pl.BlockSpec((C, C), lambda i: (0, 0)),
            pl.BlockSpec((1, C), lambda i: (0, 0)),
        ],
        out_specs=pl.BlockSpec((BLK, C), lambda i: (i, 0)),
        out_shape=jax.ShapeDtypeStruct((NP, C), jnp.float32),
    )(hc, xp, outacc,
      W1, b1.reshape(1, HIDDEN), g1.reshape(1, HIDDEN), bt1.reshape(1, HIDDEN),
      W2, b2.reshape(1, C), g2.reshape(1, C), bt2.reshape(1, C),
      W3, b3.reshape(1, C))
    return out[:N]


# speedup vs baseline: 1.7373x; 1.0920x over previous
import functools

import jax
import jax.numpy as jnp
from jax import lax
from jax.experimental import pallas as pl
from jax.experimental.pallas import tpu as pltpu
from jax.experimental.pallas import tpu_sc as plsc

N = 10000
E = 160000
C = 256
HIDDEN = 512

NC = 2
NS = 16
NW = NC * NS
B = 64
EPW = 5120
NB = EPW // B
EPAD = NW * EPW
NP = 10240
RPS = NP // NS
HW = 128
NPH = 7
ZR = 32
NZ = RPS // ZR


def _agg_body(xh, srcs, dsts, wcat, zeros,
              outacc,
              src_v, dst_v, wb, gbuf, msgbuf, acc,
              gsem0, gsem1, ssem0, ssem1):
    cid = lax.axis_index("c")
    sid = lax.axis_index("s")
    wid = cid * NS + sid

    pltpu.sync_copy(srcs.at[wid], src_v)
    pltpu.sync_copy(dsts.at[wid], dst_v)

    zero16 = jnp.zeros((16,), jnp.float32)
    iota16 = lax.iota(jnp.int32, 16)

    _dnums = lax.GatherDimensionNumbers(
        offset_dims=(), collapsed_slice_dims=(0,), start_index_map=(0,))

    def _bcast(vec16, t):
        idx = jnp.full((16, 1), t, jnp.int32)
        return lax.gather(vec16, idx, _dnums, (1,),
                          mode=lax.GatherScatterMode.PROMISE_IN_BOUNDS)

    row0 = sid * RPS
    gsems = (gsem0, gsem1)
    ssems = (ssem0, ssem1)

    def _zero_acc():
        pltpu.sync_copy(zeros, acc.at[pl.ds(row0, RPS)])

    def _dump(p):
        pltpu.sync_copy(acc.at[pl.ds(row0, RPS)],
                        outacc.at[p * NC + cid, pl.ds(row0, RPS)])

    def _issue_gather(p, g, b, j):
        half = p // 3
        cf = p - 3 * half
        pltpu.async_copy(xh.at[half].at[src_v.at[g, pl.ds(j * B, B)]],
                         gbuf.at[j], gsems[j])
        pltpu.async_copy(wcat.at[cf, wid, b], wb.at[j, pl.ds(0, B)], gsems[j])

    def _wait_gather(j):
        pltpu.make_async_copy(xh.at[0].at[src_v.at[0, pl.ds(0, B)]],
                              gbuf.at[j], gsems[j]).wait()
        pltpu.make_async_copy(wcat.at[0, 0, 0], wb.at[j, pl.ds(0, B)], gsems[j]).wait()

    def _compute(j):
        def _grp(gx, _):
            base = gx * 16
            w16 = wb[j, pl.ds(gx * 16, 16)]

            @functools.partial(plsc.parallel_loop, 0, 16, unroll=8)
            def _e2(t):
                e = base + t
                w_b = _bcast(w16, t)
                for r in range(HW // 16):
                    msgbuf[j, e, pl.ds(r * 16, 16)] = w_b * gbuf[j, e, pl.ds(r * 16, 16)]
            return ()
        lax.fori_loop(0, B // 16, _grp, ())

    def _issue_scatter(b, j):
        pltpu.async_copy(msgbuf.at[j], acc.at[dst_v.at[b]], ssems[j], add=True)

    def _wait_scatter(j):
        pltpu.make_async_copy(msgbuf.at[j], acc.at[pl.ds(0, B)], ssems[j]).wait()

    def _phase(p, _):
        plsc.subcore_barrier()
        _zero_acc()
        plsc.subcore_barrier()

        _issue_gather(p, 0, 0, 0)

        def _pair(g, _):
            b0 = g * 2
            b1 = b0 + 1
            _wait_gather(0)
            _issue_gather(p, g, b1, 1)

            @pl.when(b0 >= 2)
            def _():
                _wait_scatter(0)
            _compute(0)
            _issue_scatter(b0, 0)
            _wait_gather(1)

            @pl.when(b1 + 1 < NB)
            def _():
                _issue_gather(p, g + 1, b1 + 1, 0)

            @pl.when(b1 >= 2)
            def _():
                _wait_scatter(1)
            _compute(1)
            _issue_scatter(b1, 1)
            return ()
        lax.fori_loop(0, NB // 2, _pair, ())

        _wait_scatter(0)
        _wait_scatter(1)
        plsc.subcore_barrier()
        _dump(p)
        return ()
    lax.fori_loop(0, 6, _phase, ())

    plsc.subcore_barrier()
    _zero_acc()
    plsc.subcore_barrier()

    def _zmsg(e, _):
        for j in range(HW // 16):
            msgbuf[0, e, pl.ds(j * 16, 16)] = zero16
            msgbuf[1, e, pl.ds(j * 16, 16)] = zero16
        return ()
    lax.fori_loop(0, B, _zmsg, ())

    lane0 = iota16 == 0
    lane1 = iota16 == 1
    lane2 = iota16 == 2

    def _issue_w3(b, j):
        for kc in range(3):
            pltpu.async_copy(wcat.at[kc, wid, b], wb.at[j, pl.ds(kc * B, B)], gsems[j])

    def _wait_w3(j):
        for kc in range(3):
            pltpu.make_async_copy(wcat.at[0, 0, 0], wb.at[j, pl.ds(kc * B, B)],
                                  gsems[j]).wait()

    def _dcompute(j):
        def _dgrp(gx, _):
            base = gx * 16
            wdx16 = wb[j, pl.ds(gx * 16, 16)]
            wdy16 = wb[j, pl.ds(B + gx * 16, 16)]
            wlap16 = wb[j, pl.ds(2 * B + gx * 16, 16)]

            @functools.partial(plsc.parallel_loop, 0, 16, unroll=8)
            def _de(t):
                e = base + t
                v = jnp.where(lane0, _bcast(wdx16, t), zero16)
                v = jnp.where(lane1, _bcast(wdy16, t), v)
                v = jnp.where(lane2, _bcast(wlap16, t), v)
                msgbuf[j, e, pl.ds(0, 16)] = v
            return ()
        lax.fori_loop(0, B // 16, _dgrp, ())

    _issue_w3(0, 0)

    def _dpair(g, _):
        b0 = g * 2
        b1 = b0 + 1
        _wait_w3(0)
        _issue_w3(b1, 1)

        @pl.when(b0 >= 2)
        def _():
            _wait_scatter(0)
        _dcompute(0)
        _issue_scatter(b0, 0)

        _wait_w3(1)

        @pl.when(b1 + 1 < NB)
        def _():
            _issue_w3(b1 + 1, 0)

        @pl.when(b1 >= 2)
        def _():
            _wait_scatter(1)
        _dcompute(1)
        _issue_scatter(b1, 1)
        return ()
    lax.fori_loop(0, NB // 2, _dpair, ())

    _wait_scatter(0)
    _wait_scatter(1)
    plsc.subcore_barrier()
    _dump(6)


_agg = functools.partial(
    pl.kernel,
    out_type=[
        jax.ShapeDtypeStruct((NPH * NC, NP, HW), jnp.float32),
    ],
    mesh=plsc.VectorSubcoreMesh(core_axis_name="c", subcore_axis_name="s",
                                num_cores=NC, num_subcores=NS),
    scratch_types=[
        pltpu.VMEM((NB // 2, 2 * B), jnp.int32),
        pltpu.VMEM((NB, B), jnp.int32),
        pltpu.VMEM((2, 3 * B), jnp.float32),
        pltpu.VMEM((2, B, HW), jnp.float32),
        pltpu.VMEM((2, B, HW), jnp.float32),
        pltpu.VMEM_SHARED((NP, HW), jnp.float32),
        pltpu.SemaphoreType.DMA,
        pltpu.SemaphoreType.DMA,
        pltpu.SemaphoreType.DMA,
        pltpu.SemaphoreType.DMA,
    ],
)(_agg_body)


BLK = 128


def _mlp_body(hc_ref, x_ref, acc_ref,
              W1_ref, b1_ref, g1_ref, bt1_ref,
              W2_ref, b2_ref, g2_ref, bt2_ref,
              W3_ref, b3_ref, o_ref):
    h = hc_ref[0, 0]
    xb = x_ref[...]
    acct = acc_ref[...]
    degc = acct[12] + acct[13]

    inv_sqrt2 = 0.7071067811865476

    def gelu(v):
        return 0.5 * v * (1.0 + lax.erf(v * inv_sqrt2))

    def ln(v, g, b):
        mu = jnp.mean(v, axis=-1, keepdims=True)
        var = jnp.mean((v - mu) ** 2, axis=-1, keepdims=True)
        return (v - mu) * lax.rsqrt(var + 1e-5) * g + b

    hs = [h, h, h * h]
    z = xb @ W1_ref[0:C]
    for kc in range(3):
        Y = jnp.concatenate(
            [acct[2 * kc] + acct[2 * kc + 1],
             acct[6 + 2 * kc] + acct[6 + 2 * kc + 1]], axis=1)
        deg = degc[:, kc:kc + 1]
        feat = (Y - deg * xb) * hs[kc]
        z = z + feat @ W1_ref[(kc + 1) * C:(kc + 2) * C]
    z = z + b1_ref[...]
    z = gelu(ln(z, g1_ref[...], bt1_ref[...]))
    z = z @ W2_ref[...] + b2_ref[...]
    z = gelu(ln(z, g2_ref[...], bt2_ref[...]))
    o_ref[...] = z @ W3_ref[...] + b3_ref[...] + xb


def kernel(x, edge_index, coeff_dx, coeff_dy, coeff_lap, h_char,
           W1, b1, g1, bt1, W2, b2, g2, bt2, W3, b3):
    src = edge_index[0]
    dst = edge_index[1]
    pad = EPAD - E
    srcp = jnp.concatenate([src, jnp.zeros((pad,), jnp.int32)]).reshape(NW, NB // 2, 2 * B)
    dstp = jnp.concatenate([dst, jnp.zeros((pad,), jnp.int32)]).reshape(NW, NB, B)
    zpadf = jnp.zeros((pad,), jnp.float32)
    wcat = jnp.stack([
        jnp.concatenate([coeff_dx.reshape(E), zpadf]).reshape(NW, NB, B),
        jnp.concatenate([coeff_dy.reshape(E), zpadf]).reshape(NW, NB, B),
        jnp.concatenate([coeff_lap.reshape(E), zpadf]).reshape(NW, NB, B),
    ])

    xh = jnp.stack([x[:, :HW], x[:, HW:]])
    zeros = jnp.zeros((RPS, HW), jnp.float32)

    (outacc,) = _agg(xh, srcp, dstp, wcat, zeros)

    xp = jnp.pad(x, ((0, NP - N), (0, 0)))
    hc = h_char.reshape(1, 1)

    grid = (NP // BLK,)
    out = pl.pallas_call(
        _mlp_body,
        grid=grid,
        in_specs=[
            pl.BlockSpec(memory_space=pltpu.SMEM),
            pl.BlockSpec((BLK, C), lambda i: (i, 0)),
            pl.BlockSpec((NPH * NC, BLK, HW), lambda i: (0, i, 0)),
            pl.BlockSpec((4 * C, HIDDEN), lambda i: (0, 0)),
            pl.BlockSpec((1, HIDDEN), lambda i: (0, 0)),
            pl.BlockSpec((1, HIDDEN), lambda i: (0, 0)),
            pl.BlockSpec((1, HIDDEN), lambda i: (0, 0)),
            pl.BlockSpec((HIDDEN, C), lambda i: (0, 0)),
            pl.BlockSpec((1, C), lambda i: (0, 0)),
            pl.BlockSpec((1, C), lambda i: (0, 0)),
            pl.BlockSpec((1, C), lambda i: (0, 0)),
            pl.BlockSpec((C, C), lambda i: (0, 0)),
            pl.BlockSpec((1, C), lambda i: (0, 0)),
        ],
        out_specs=pl.BlockSpec((BLK, C), lambda i: (i, 0)),
        out_shape=jax.ShapeDtypeStruct((NP, C), jnp.float32),
    )(hc, xp, outacc,
      W1, b1.reshape(1, HIDDEN), g1.reshape(1, HIDDEN), bt1.reshape(1, HIDDEN),
      W2, b2.reshape(1, C), g2.reshape(1, C), bt2.reshape(1, C),
      W3, b3.reshape(1, C))
    return out[:N]
